# baseline (speedup 1.0000x reference)
import jax
import jax.numpy as jnp
from jax import lax
from jax.experimental import pallas as pl
from jax.experimental.pallas import tpu as pltpu
from jax.experimental.pallas import tpu_sc as plsc

BATCH = 1024
SEQ = 200
EMBED_DIM = 64
SCALE = 8.0
VOCAB_ROWS = 1000000

NUM_CORES = 2
NUM_SUBCORES = 16
NW = NUM_CORES * NUM_SUBCORES
SEQ_PER_W = BATCH // NW
ROWS_PER_W = SEQ_PER_W * SEQ
SPLIT = 96
NBUF = 4
VREGS_PER_ROW = EMBED_DIM // 16


def _pe_kernel_body(src_hbm, pe_hbm, table_hbm, out_hbm, idx_v, pe_v, *bufs):
    ins = bufs[0:NBUF]
    outs = bufs[NBUF : 2 * NBUF]
    sin = bufs[2 * NBUF : 3 * NBUF]
    sout = bufs[3 * NBUF : 4 * NBUF]

    wid = lax.axis_index("s") * NUM_CORES + lax.axis_index("c")
    bbase = wid * SEQ_PER_W

    pltpu.sync_copy(src_hbm.at[wid], idx_v)
    pltpu.sync_copy(pe_hbm, pe_v)

    def start_gathers(s, b):
        o = s * SEQ
        pltpu.async_copy(
            table_hbm.at[idx_v.at[pl.ds(o, SPLIT)]], ins[b].at[pl.ds(0, SPLIT)], sin[b]
        )
        pltpu.async_copy(
            table_hbm.at[idx_v.at[pl.ds(o + SPLIT, SEQ - SPLIT)]],
            ins[b].at[pl.ds(SPLIT, SEQ - SPLIT)],
            sin[b],
        )

    def wait_gathers(s, b):
        o = s * SEQ
        pltpu.make_async_copy(
            table_hbm.at[idx_v.at[pl.ds(o, SPLIT)]], ins[b].at[pl.ds(0, SPLIT)], sin[b]
        ).wait()
        pltpu.make_async_copy(
            table_hbm.at[idx_v.at[pl.ds(o + SPLIT, SEQ - SPLIT)]],
            ins[b].at[pl.ds(SPLIT, SEQ - SPLIT)],
            sin[b],
        ).wait()

    for b in range(NBUF):
        start_gathers(b, b)

    def block_body(g0, _):
        g = g0 * NBUF
        for b in range(NBUF):
            s = g + b
            wait_gathers(s, b)
            @pl.when(s >= NBUF)
            def _():
                pltpu.make_async_copy(out_hbm.at[0], outs[b], sout[b]).wait()

            def row_body(r, _):
                for d in range(VREGS_PER_ROW):
                    o = d * 16
                    outs[b][r, pl.ds(o, 16)] = (
                        ins[b][r, pl.ds(o, 16)] * SCALE + pe_v[r, pl.ds(o, 16)]
                    )
                return ()

            lax.fori_loop(0, SEQ, row_body, (), unroll=4)

            pltpu.async_copy(outs[b], out_hbm.at[bbase + s], sout[b])

            @pl.when(s + NBUF < SEQ_PER_W)
            def _():
                start_gathers(s + NBUF, b)

        return ()

    lax.fori_loop(0, SEQ_PER_W // NBUF, block_body, ())

    for b in range(NBUF):
        pltpu.make_async_copy(out_hbm.at[0], outs[b], sout[b]).wait()


@jax.jit
def kernel(src, table, pe):
    src_r = src.reshape(NW, ROWS_PER_W)
    pe_seq = pe[:SEQ]
    tableL = lax.optimization_barrier(
        table.reshape(VOCAB_ROWS * EMBED_DIM)
    ).reshape(VOCAB_ROWS, EMBED_DIM)

    mesh = plsc.VectorSubcoreMesh(core_axis_name="c", subcore_axis_name="s")
    out = pl.kernel(
        _pe_kernel_body,
        out_type=jax.ShapeDtypeStruct((BATCH, SEQ, EMBED_DIM), jnp.float32),
        mesh=mesh,
        compiler_params=pltpu.CompilerParams(use_tc_tiling_on_sc=False),
        scratch_types=(
            [
                pltpu.VMEM((ROWS_PER_W,), jnp.int32),
                pltpu.VMEM((SEQ, EMBED_DIM), jnp.float32),
            ]
            + [pltpu.VMEM((SEQ, EMBED_DIM), jnp.float32) for _ in range(2 * NBUF)]
            + [pltpu.SemaphoreType.DMA for _ in range(2 * NBUF)]
        ),
    )(src_r, pe_seq, tableL)
    return out
